# baseline (speedup 1.0000x reference)
import math

import jax
import jax.numpy as jnp
from jax.experimental import pallas as pl
from jax.experimental.pallas import tpu as pltpu


def _pab_kernel(qt_ref, k_ref, vt_ref, e_ref, wqv_ref, wk_ref, o_ref):
    ch = k_ref.shape[1]
    T = k_ref.shape[2]
    scale = 1.0 / math.sqrt(math.sqrt(ch))
    e = e_ref[0]
    dn_nn = (((1,), (0,)), ((), ()))
    qv_null_t = jax.lax.dot_general(
        e, wqv_ref[...], dn_nn, preferred_element_type=jnp.float32)
    k_null = jax.lax.dot_general(
        wk_ref[...], e, (((1,), (1,)), ((), ())),
        preferred_element_type=jnp.float32)
    qe_t = (qt_ref[0] + qv_null_t[:, 0:ch]) * scale
    ke = (k_ref[0] + k_null) * scale
    ve_t = vt_ref[0] + qv_null_t[:, ch:2 * ch]
    logits = jax.lax.dot_general(
        qe_t, ke, dn_nn, preferred_element_type=jnp.float32)
    ew = jnp.exp(logits)
    a_t = jax.lax.dot_general(
        ew, ve_t, dn_nn, preferred_element_type=jnp.float32)
    ones = jnp.ones((T, 8), dtype=jnp.float32)
    sums = jax.lax.dot_general(
        ew, ones, dn_nn, preferred_element_type=jnp.float32)
    o_ref[0] = a_t / sums[:, 0:1]


def kernel(qkv, cls_embedding, W_cls):
    bs, width, T = qkv.shape
    n_heads = 16
    ch = width // (3 * n_heads)
    B = bs * n_heads
    E = cls_embedding.shape[2]
    qkv_r = qkv.reshape(B, 3 * ch, T)
    q_t = qkv_r[:, 0:ch, :].transpose(0, 2, 1)
    k = qkv_r[:, ch:2 * ch, :]
    v_t = qkv_r[:, 2 * ch:3 * ch, :].transpose(0, 2, 1)
    w_t = W_cls.T
    wqv = jnp.concatenate([w_t[:, 0:ch], w_t[:, 2 * ch:3 * ch]], axis=1)
    wk = W_cls[ch:2 * ch, :]
    out_t = pl.pallas_call(
        _pab_kernel,
        grid=(B,),
        in_specs=[
            pl.BlockSpec((1, T, ch), lambda b: (b, 0, 0)),
            pl.BlockSpec((1, ch, T), lambda b: (b, 0, 0)),
            pl.BlockSpec((1, T, ch), lambda b: (b, 0, 0)),
            pl.BlockSpec((1, T, E), lambda b: (b, 0, 0)),
            pl.BlockSpec((E, 2 * ch), lambda b: (0, 0)),
            pl.BlockSpec((ch, E), lambda b: (0, 0)),
        ],
        out_specs=pl.BlockSpec((1, T, ch), lambda b: (b, 0, 0)),
        out_shape=jax.ShapeDtypeStruct((B, T, ch), qkv.dtype),
    )(q_t, k, v_t, cls_embedding, wqv, wk)
    return out_t.transpose(0, 2, 1).reshape(bs, n_heads * ch, T)


# speedup vs baseline: 2.0030x; 2.0030x over previous
import math

import jax
import jax.numpy as jnp
from jax.experimental import pallas as pl
from jax.experimental.pallas import tpu as pltpu


def _pab_kernel(qkv_ref, e_ref, w_ref, o_ref):
    ch = o_ref.shape[1]
    T = o_ref.shape[2]
    scale = 1.0 / math.sqrt(math.sqrt(ch))
    e = e_ref[0]
    dn_te = (((1,), (1,)), ((), ()))
    null = jax.lax.dot_general(
        w_ref[...], e, dn_te, preferred_element_type=jnp.float32)
    qe = (qkv_ref[0, 0:ch, :] + null[0:ch, :]) * scale
    ke = (qkv_ref[0, ch:2 * ch, :] + null[ch:2 * ch, :]) * scale
    ve = qkv_ref[0, 2 * ch:3 * ch, :] + null[2 * ch:3 * ch, :]
    logits = jax.lax.dot_general(
        qe, ke, (((0,), (0,)), ((), ())),
        preferred_element_type=jnp.float32)
    ew = jnp.exp(logits)
    ve_aug = jnp.concatenate(
        [ve.T, jnp.ones((T, 8), dtype=jnp.float32)], axis=1)
    a_aug = jax.lax.dot_general(
        ew, ve_aug, (((1,), (0,)), ((), ())),
        preferred_element_type=jnp.float32)
    a_t = a_aug[:, 0:ch] / a_aug[:, ch:ch + 1]
    o_ref[0] = a_t.T


def kernel(qkv, cls_embedding, W_cls):
    bs, width, T = qkv.shape
    n_heads = 16
    ch = width // (3 * n_heads)
    B = bs * n_heads
    E = cls_embedding.shape[2]
    qkv_r = qkv.reshape(B, 3 * ch, T)
    out = pl.pallas_call(
        _pab_kernel,
        grid=(B,),
        in_specs=[
            pl.BlockSpec((1, 3 * ch, T), lambda b: (b, 0, 0)),
            pl.BlockSpec((1, T, E), lambda b: (b, 0, 0)),
            pl.BlockSpec((3 * ch, E), lambda b: (0, 0)),
        ],
        out_specs=pl.BlockSpec((1, ch, T), lambda b: (b, 0, 0)),
        out_shape=jax.ShapeDtypeStruct((B, ch, T), qkv.dtype),
    )(qkv_r, cls_embedding, W_cls)
    return out.reshape(bs, n_heads * ch, T)
